# baseline (speedup 1.0000x reference)
import jax
import jax.numpy as jnp
from jax import lax
from jax.experimental import pallas as pl
from jax.experimental.pallas import tpu as pltpu


def _body(mask_ref, conf_ref, feat_ref, anc_ref, cfeat_ref, canc_ref,
          outf_ref, outa_ref, topk_ref):
    b = pl.program_id(0)
    N = conf_ref.shape[1]
    T = cfeat_ref.shape[1]
    K = N - T

    conf = jnp.max(conf_ref[0], axis=-1)
    col = conf[:, None]
    row = conf[None, :]
    jj = lax.broadcasted_iota(jnp.int32, (N, N), 0)
    ii = lax.broadcasted_iota(jnp.int32, (N, N), 1)
    gt = (col > row) | ((col == row) & (jj < ii))
    rank = jnp.sum(gt.astype(jnp.int32), axis=0)

    r_iota = lax.broadcasted_iota(jnp.int32, (N, K), 1)
    E = (rank[:, None] == r_iota).astype(jnp.float32)

    topk_vals = jnp.sum(E * conf[:, None], axis=0)
    sel_feat = lax.dot_general(E, feat_ref[0], (((0,), (0,)), ((), ())),
                               preferred_element_type=jnp.float32,
                               precision=lax.Precision.HIGHEST)
    sel_anc = lax.dot_general(E, anc_ref[0], (((0,), (0,)), ((), ())),
                              preferred_element_type=jnp.float32,
                              precision=lax.Precision.HIGHEST)

    m = mask_ref[b] != 0
    outf_ref[0, :T] = jnp.where(m, cfeat_ref[0], feat_ref[0, :T])
    outf_ref[0, T:] = jnp.where(m, sel_feat, feat_ref[0, T:])
    outa_ref[0, :T] = jnp.where(m, canc_ref[0], anc_ref[0, :T])
    outa_ref[0, T:] = jnp.where(m, sel_anc, anc_ref[0, T:])
    topk_ref[0, 0] = topk_vals


def kernel(confidence, instance_feature, anchor, cached_feature,
           cached_anchor, mask):
    bs, N, C = confidence.shape
    D = instance_feature.shape[2]
    A = anchor.shape[2]
    T = cached_feature.shape[1]
    K = N - T

    mask_i32 = mask.astype(jnp.int32)

    grid = (bs,)
    out_shapes = (
        jax.ShapeDtypeStruct((bs, N, D), jnp.float32),
        jax.ShapeDtypeStruct((bs, N, A), jnp.float32),
        jax.ShapeDtypeStruct((bs, 1, K), jnp.float32),
    )
    in_specs = [
        pl.BlockSpec(memory_space=pltpu.SMEM),
        pl.BlockSpec((1, N, C), lambda b: (b, 0, 0)),
        pl.BlockSpec((1, N, D), lambda b: (b, 0, 0)),
        pl.BlockSpec((1, N, A), lambda b: (b, 0, 0)),
        pl.BlockSpec((1, T, D), lambda b: (b, 0, 0)),
        pl.BlockSpec((1, T, A), lambda b: (b, 0, 0)),
    ]
    out_specs = (
        pl.BlockSpec((1, N, D), lambda b: (b, 0, 0)),
        pl.BlockSpec((1, N, A), lambda b: (b, 0, 0)),
        pl.BlockSpec((1, 1, K), lambda b: (b, 0, 0)),
    )
    outf, outa, topk = pl.pallas_call(
        _body,
        grid=grid,
        in_specs=in_specs,
        out_specs=out_specs,
        out_shape=out_shapes,
    )(mask_i32, confidence, instance_feature, anchor, cached_feature,
      cached_anchor)
    return outf, outa, topk.reshape(bs, K)
